# baseline (speedup 1.0000x reference)
import functools

import jax
import jax.numpy as jnp
from jax import lax
from jax.experimental import pallas as pl
from jax.experimental.pallas import tpu as pltpu
from jax.experimental.pallas import tpu_sc as plsc

B = 64
N = 512
P = 8
ZD = 256
TOPK = 256
TOT = N * P
NVREG = TOT // 16
import numpy as np

NEGINF = np.float32(-np.inf)
BIG = np.int32(1 << 30)


def _lanes():
    return lax.iota(jnp.int32, 16)


def _sel_body(att_hbm, z_hbm, mu_hbm, sig_hbm,
              ztopk_hbm, mutopk_hbm, sigtopk_hbm,
              att_v, key_v, key0_v, l1k_v, l1p_v, pos_v, idx_v,
              rows_v, sem):
    wid = lax.axis_index("s") * 2 + lax.axis_index("c")
    lanes = _lanes()
    bags = (wid * 2, wid * 2 + 1)

    for t in range(2):
        pltpu.sync_copy(att_hbm.at[bags[t]], att_v.at[pl.ds(t * TOT, TOT)])

    def p1(g, _):
        for t in range(2):
            l1k = jnp.full((16,), NEGINF, jnp.float32)
            l1p = jnp.full((16,), BIG, jnp.int32)
            for j in range(16):
                i = g * 16 + j
                key = att_v[pl.ds(t * TOT + i * 16, 16)]
                key_v[pl.ds(t * TOT + i * 16, 16)] = key
                m = jnp.max(key)
                lp = jnp.min(jnp.where(key == m, lanes, jnp.int32(16)))
                l1k = jnp.where(lanes == j, m, l1k)
                l1p = jnp.where(lanes == j, i * 16 + lp, l1p)
            l1k_v[pl.ds(t * NVREG + g * 16, 16)] = l1k
            l1p_v[pl.ds(t * NVREG + g * 16, 16)] = l1p
        return 0

    lax.fori_loop(0, 16, p1, 0)

    def p2(g, carry):
        out = []
        for t in range(2):
            l2k, l2p = carry[2 * t], carry[2 * t + 1]
            gk = l1k_v[pl.ds(t * NVREG + g * 16, 16)]
            gp = l1p_v[pl.ds(t * NVREG + g * 16, 16)]
            m = jnp.max(gk)
            pos = jnp.min(jnp.where(gk == m, gp, BIG))
            out.append(jnp.where(lanes == g, m, l2k))
            out.append(jnp.where(lanes == g, pos, l2p))
        return tuple(out)

    ik = jnp.full((16,), NEGINF, jnp.float32)
    ip = jnp.full((16,), BIG, jnp.int32)
    l2 = lax.fori_loop(0, 16, p2, (ik, ip, ik, ip))

    def p3(o, carry):
        l2k0, l2p0, l2k1, l2p1, tk0, tk1 = carry
        st = [[l2k0, l2p0, tk0, jnp.full((16,), 0, jnp.int32)],
              [l2k1, l2p1, tk1, jnp.full((16,), 0, jnp.int32)]]
        for j in range(16):
            for t in range(2):
                l2k, l2p, tkey, posacc = st[t]
                m2 = jnp.max(l2k)
                win = jnp.min(jnp.where(l2k == m2, l2p, BIG))
                posacc = jnp.where(lanes == j, win, posacc)
                vidx = win // 16
                lane = win % 16
                base = t * TOT + vidx * 16
                vec = key_v[pl.ds(base, 16)]
                vec = jnp.where(lanes == lane, NEGINF, vec)
                key_v[pl.ds(base, 16)] = vec
                m1 = jnp.max(vec)
                p1n = vidx * 16 + jnp.min(jnp.where(vec == m1, lanes, jnp.int32(16)))
                grp = vidx // 16
                lane2 = vidx % 16
                gbase = t * NVREG + grp * 16
                gk = l1k_v[pl.ds(gbase, 16)]
                gp = l1p_v[pl.ds(gbase, 16)]
                gk = jnp.where(lanes == lane2, m1, gk)
                gp = jnp.where(lanes == lane2, jnp.where(m1 == NEGINF, BIG, p1n), gp)
                l1k_v[pl.ds(gbase, 16)] = gk
                l1p_v[pl.ds(gbase, 16)] = gp
                m2n = jnp.max(gk)
                p2n = jnp.min(jnp.where(gk == m2n, gp, BIG))
                st[t] = [jnp.where(lanes == grp, m2n, l2k),
                         jnp.where(lanes == grp, p2n, l2p),
                         m2, posacc]
        for t in range(2):
            pos_v[pl.ds(t * TOPK + o * 16, 16)] = st[t][3]
        return st[0][0], st[0][1], st[1][0], st[1][1], st[0][2], st[1][2]

    res = lax.fori_loop(0, 16, p3, l2 + (NEGINF, NEGINF))
    tkeys = (res[4], res[5])

    idx1_offs = (lanes % 8) * 512 + lanes // 8

    def cgt_body(i, carry):
        a0, a1 = carry
        k0 = key0_v[pl.ds(i * 16, 16)]
        k1 = key0_v[pl.ds(TOT + i * 16, 16)]
        a0 = a0 + jnp.where(k0 > tkeys[0], 1, 0).astype(jnp.int32)
        a1 = a1 + jnp.where(k1 > tkeys[1], 1, 0).astype(jnp.int32)
        return a0, a1

    zero = jnp.zeros((16,), jnp.int32)
    cg0, cg1 = lax.fori_loop(0, NVREG, cgt_body, (zero, zero))
    r_all = (jnp.int32(TOPK) - jnp.sum(cg0), jnp.int32(TOPK) - jnp.sum(cg1))

    def p4a(k, carry):
        run0, tie0, run1, tie1 = carry
        pos16 = idx1_offs + 2 * k
        out = []
        for t, run, tie in ((0, run0, tie0), (1, run1, tie1)):
            key = plsc.load_gather(key0_v, [pos16 + t * TOT])
            eq = key == tkeys[t]
            cum = lax.cumsum(jnp.where(eq, 1, 0).astype(jnp.int32), axis=0)
            sel = jnp.where(eq, (run + cum) <= r_all[t], False)
            out.append(run + jnp.max(cum))
            out.append(tie + jnp.where(sel, 1, 0).astype(jnp.int32))
        return out[0], out[1], out[2], out[3]

    _, tie0, _, tie1 = lax.fori_loop(
        0, NVREG, p4a, (jnp.int32(0), zero, jnp.int32(0), zero))
    ties = (tie0, tie1)

    pstars = []
    for t in range(2):
        cnts = jnp.where(lanes < P, 0, -1).astype(jnp.int32)
        for p in range(P):
            def prow(i, acc):
                key = key0_v[pl.ds(t * TOT + p * N + i * 16, 16)]
                return acc + jnp.where(key > tkeys[t], 1, 0).astype(jnp.int32)

            acc = lax.fori_loop(0, N // 16, prow, zero)
            tp = jnp.sum(jnp.where((lanes == p) | (lanes == p + 8), ties[t], 0))
            cnts = jnp.where(lanes == p, jnp.sum(acc) + tp, cnts)
        mx = jnp.max(cnts)
        pstars.append(jnp.min(jnp.where(cnts == mx, lanes, jnp.int32(16))))

    for t in range(2):
        pltpu.sync_copy(mu_hbm.at[pstars[t]], mutopk_hbm.at[bags[t]])
        pltpu.sync_copy(sig_hbm.at[pstars[t]], sigtopk_hbm.at[bags[t]])

    def p6(o, _):
        for t in range(2):
            vec = pos_v[pl.ds(t * TOPK + o * 16, 16)]
            idx_v[pl.ds(t * TOPK + o * 16, 16)] = bags[t] * N + (vec & (N - 1))
        return 0

    lax.fori_loop(0, 16, p6, 0)
    for t in range(2):
        for c in range(2):
            pltpu.async_copy(z_hbm.at[idx_v.at[pl.ds(t * TOPK + c * 128, 128)]],
                             rows_v, sem).wait()
            pltpu.sync_copy(rows_v, ztopk_hbm.at[bags[t], pl.ds(c * 128, 128)])


@functools.partial(
    pl.kernel,
    out_type=[
        jax.ShapeDtypeStruct((B, TOPK, ZD), jnp.float32),
        jax.ShapeDtypeStruct((B, ZD), jnp.float32),
        jax.ShapeDtypeStruct((B, ZD), jnp.float32),
    ],
    mesh=plsc.VectorSubcoreMesh(core_axis_name="c", subcore_axis_name="s"),
    compiler_params=pltpu.CompilerParams(needs_layout_passes=False),
    scratch_types=[
        pltpu.VMEM((2 * TOT,), jnp.float32),
        pltpu.VMEM((2 * TOT,), jnp.float32),
        pltpu.VMEM((2 * NVREG,), jnp.float32),
        pltpu.VMEM((2 * NVREG,), jnp.int32),
        pltpu.VMEM((2 * TOPK,), jnp.int32),
        pltpu.VMEM((2 * TOPK,), jnp.int32),
        pltpu.VMEM((128, ZD), jnp.float32),
        pltpu.SemaphoreType.DMA,
    ],
)
def _sel_kernel(att_hbm, z_hbm, mu_hbm, sig_hbm,
                ztopk_hbm, mutopk_hbm, sigtopk_hbm,
                att_v, key_v, l1k_v, l1p_v, pos_v, idx_v,
                rows_v, sem):
    _sel_body(att_hbm, z_hbm, mu_hbm, sig_hbm,
              ztopk_hbm, mutopk_hbm, sigtopk_hbm,
              att_v, key_v, att_v, l1k_v, l1p_v, pos_v, idx_v,
              rows_v, sem)


def _dec_body(zs_ref, wd_ref, bd_ref, o_ref):
    acc = jnp.dot(zs_ref[...], wd_ref[...], preferred_element_type=jnp.float32)
    acc = acc + bd_ref[...]
    o_ref[...] = acc


def _decoder_logits(z_proxy_sample, Wd, bd):
    nproxy, S, zdim = z_proxy_sample.shape
    C = Wd.shape[1]
    flat = z_proxy_sample.reshape(nproxy * S, zdim)
    out = pl.pallas_call(
        _dec_body,
        out_shape=jax.ShapeDtypeStruct((nproxy * S, C), jnp.float32),
    )(flat, Wd, bd.reshape(1, -1))
    return jnp.mean(out.reshape(nproxy, S, C), axis=1)


def kernel(x, W1, b1, W2, b2, W3, b3, Wd, bd, proxies, eps_proxy, topk):
    zdim = W3.shape[1]

    mu_proxy = proxies[:, :zdim]
    sigma_proxy = jax.nn.softplus(proxies[:, zdim:])
    z_proxy_sample = mu_proxy[:, None, :] + sigma_proxy[:, None, :] * eps_proxy
    z_proxy = jnp.mean(z_proxy_sample, axis=1)

    def _l2norm(v, axis):
        n = jnp.sqrt(jnp.sum(v * v, axis=axis, keepdims=True))
        return v / jnp.maximum(n, 1e-12)

    z_proxy_norm = _l2norm(z_proxy, axis=1)

    h = jnp.maximum(jnp.einsum('bnf,fg->bng', x, W1) + b1, 0.0)
    h = jnp.maximum(jnp.einsum('bng,gh->bnh', h, W2) + b2, 0.0)
    z = jnp.einsum('bnh,hz->bnz', h, W3) + b3
    z_norm = _l2norm(z, axis=2)
    att = jnp.einsum('bnz,pz->bnp', z_norm, z_proxy_norm)

    z, att = jax.lax.optimization_barrier((z, att))

    att_t = jnp.transpose(att, (0, 2, 1)).reshape(B, TOT)
    z_flat = z.reshape(B * N, zdim)

    z_topk, mu_topk, sigma_topk = _sel_kernel(
        att_t, z_flat, mu_proxy, sigma_proxy)

    decoder_logits_proxy = _decoder_logits(z_proxy_sample, Wd, bd)
    return (decoder_logits_proxy, mu_proxy, sigma_proxy,
            z_topk, mu_topk, sigma_topk)


# speedup vs baseline: 1.0041x; 1.0041x over previous
import functools

import jax
import jax.numpy as jnp
from jax import lax
from jax.experimental import pallas as pl
from jax.experimental.pallas import tpu as pltpu
from jax.experimental.pallas import tpu_sc as plsc

B = 64
N = 512
P = 8
ZD = 256
TOPK = 256
TOT = N * P
NVREG = TOT // 16
import numpy as np

NEGINF = np.float32(-np.inf)
BIG = np.int32(1 << 30)


def _lanes():
    return lax.iota(jnp.int32, 16)


def _sel_body(att_hbm, z_hbm, mu_hbm, sig_hbm,
              ztopk_hbm, mutopk_hbm, sigtopk_hbm,
              atts, keys, l1ks, l1ps, poss, idx_v,
              rows_v, sem):
    wid = lax.axis_index("s") * 2 + lax.axis_index("c")
    lanes = _lanes()
    bags = (wid * 2, wid * 2 + 1)

    for t in range(2):
        pltpu.sync_copy(att_hbm.at[bags[t]], atts[t])

    def p1(g, _):
        for t in range(2):
            l1k = jnp.full((16,), NEGINF, jnp.float32)
            l1p = jnp.full((16,), BIG, jnp.int32)
            for j in range(16):
                i = g * 16 + j
                key = atts[t][pl.ds(i * 16, 16)]
                keys[t][pl.ds(i * 16, 16)] = key
                m = jnp.max(key)
                lp = jnp.min(jnp.where(key == m, lanes, jnp.int32(16)))
                l1k = jnp.where(lanes == j, m, l1k)
                l1p = jnp.where(lanes == j, i * 16 + lp, l1p)
            l1ks[t][pl.ds(g * 16, 16)] = l1k
            l1ps[t][pl.ds(g * 16, 16)] = l1p
        return 0

    lax.fori_loop(0, 16, p1, 0)

    def p2(g, carry):
        out = []
        for t in range(2):
            l2k, l2p = carry[2 * t], carry[2 * t + 1]
            gk = l1ks[t][pl.ds(g * 16, 16)]
            gp = l1ps[t][pl.ds(g * 16, 16)]
            m = jnp.max(gk)
            pos = jnp.min(jnp.where(gk == m, gp, BIG))
            out.append(jnp.where(lanes == g, m, l2k))
            out.append(jnp.where(lanes == g, pos, l2p))
        return tuple(out)

    ik = jnp.full((16,), NEGINF, jnp.float32)
    ip = jnp.full((16,), BIG, jnp.int32)
    l2 = lax.fori_loop(0, 16, p2, (ik, ip, ik, ip))

    def p3(o, carry):
        l2k0, l2p0, l2k1, l2p1, tk0, tk1 = carry
        st = [[l2k0, l2p0, tk0, jnp.full((16,), 0, jnp.int32)],
              [l2k1, l2p1, tk1, jnp.full((16,), 0, jnp.int32)]]
        for j in range(16):
            for t in range(2):
                l2k, l2p, tkey, posacc = st[t]
                m2 = jnp.max(l2k)
                win = jnp.min(jnp.where(l2k == m2, l2p, BIG))
                posacc = jnp.where(lanes == j, win, posacc)
                vidx = win // 16
                lane = win % 16
                vec = keys[t][pl.ds(vidx * 16, 16)]
                vec = jnp.where(lanes == lane, NEGINF, vec)
                keys[t][pl.ds(vidx * 16, 16)] = vec
                m1 = jnp.max(vec)
                p1n = vidx * 16 + jnp.min(jnp.where(vec == m1, lanes, jnp.int32(16)))
                grp = vidx // 16
                lane2 = vidx % 16
                gk = l1ks[t][pl.ds(grp * 16, 16)]
                gp = l1ps[t][pl.ds(grp * 16, 16)]
                gk = jnp.where(lanes == lane2, m1, gk)
                gp = jnp.where(lanes == lane2, jnp.where(m1 == NEGINF, BIG, p1n), gp)
                l1ks[t][pl.ds(grp * 16, 16)] = gk
                l1ps[t][pl.ds(grp * 16, 16)] = gp
                m2n = jnp.max(gk)
                p2n = jnp.min(jnp.where(gk == m2n, gp, BIG))
                st[t] = [jnp.where(lanes == grp, m2n, l2k),
                         jnp.where(lanes == grp, p2n, l2p),
                         m2, posacc]
        for t in range(2):
            poss[t][pl.ds(o * 16, 16)] = st[t][3]
        return st[0][0], st[0][1], st[1][0], st[1][1], st[0][2], st[1][2]

    res = lax.fori_loop(0, 16, p3, l2 + (NEGINF, NEGINF))
    tkeys = (res[4], res[5])

    idx1_offs = (lanes % 8) * 512 + lanes // 8

    def cgt_body(i, carry):
        a0, a1 = carry
        k0 = atts[0][pl.ds(i * 16, 16)]
        k1 = atts[1][pl.ds(i * 16, 16)]
        a0 = a0 + jnp.where(k0 > tkeys[0], 1, 0).astype(jnp.int32)
        a1 = a1 + jnp.where(k1 > tkeys[1], 1, 0).astype(jnp.int32)
        return a0, a1

    zero = jnp.zeros((16,), jnp.int32)
    cg0, cg1 = lax.fori_loop(0, NVREG, cgt_body, (zero, zero))
    r_all = (jnp.int32(TOPK) - jnp.sum(cg0), jnp.int32(TOPK) - jnp.sum(cg1))

    def p4a(k, carry):
        run0, tie0, run1, tie1 = carry
        pos16 = idx1_offs + 2 * k
        out = []
        for t, run, tie in ((0, run0, tie0), (1, run1, tie1)):
            key = plsc.load_gather(atts[t], [pos16])
            eq = key == tkeys[t]
            cum = lax.cumsum(jnp.where(eq, 1, 0).astype(jnp.int32), axis=0)
            sel = jnp.where(eq, (run + cum) <= r_all[t], False)
            out.append(run + jnp.max(cum))
            out.append(tie + jnp.where(sel, 1, 0).astype(jnp.int32))
        return out[0], out[1], out[2], out[3]

    _, tie0, _, tie1 = lax.fori_loop(
        0, NVREG, p4a, (jnp.int32(0), zero, jnp.int32(0), zero))
    ties = (tie0, tie1)

    pstars = []
    for t in range(2):
        cnts = jnp.where(lanes < P, 0, -1).astype(jnp.int32)
        for p in range(P):
            def prow(i, acc):
                key = atts[t][pl.ds(p * N + i * 16, 16)]
                return acc + jnp.where(key > tkeys[t], 1, 0).astype(jnp.int32)

            acc = lax.fori_loop(0, N // 16, prow, zero)
            tp = jnp.sum(jnp.where((lanes == p) | (lanes == p + 8), ties[t], 0))
            cnts = jnp.where(lanes == p, jnp.sum(acc) + tp, cnts)
        mx = jnp.max(cnts)
        pstars.append(jnp.min(jnp.where(cnts == mx, lanes, jnp.int32(16))))

    for t in range(2):
        pltpu.sync_copy(mu_hbm.at[pstars[t]], mutopk_hbm.at[bags[t]])
        pltpu.sync_copy(sig_hbm.at[pstars[t]], sigtopk_hbm.at[bags[t]])

    def p6(o, _):
        for t in range(2):
            vec = poss[t][pl.ds(o * 16, 16)]
            idx_v[pl.ds(t * TOPK + o * 16, 16)] = bags[t] * N + (vec & (N - 1))
        return 0

    lax.fori_loop(0, 16, p6, 0)
    for t in range(2):
        for c in range(2):
            pltpu.async_copy(z_hbm.at[idx_v.at[pl.ds(t * TOPK + c * 128, 128)]],
                             rows_v, sem).wait()
            pltpu.sync_copy(rows_v, ztopk_hbm.at[bags[t], pl.ds(c * 128, 128)])


@functools.partial(
    pl.kernel,
    out_type=[
        jax.ShapeDtypeStruct((B, TOPK, ZD), jnp.float32),
        jax.ShapeDtypeStruct((B, ZD), jnp.float32),
        jax.ShapeDtypeStruct((B, ZD), jnp.float32),
    ],
    mesh=plsc.VectorSubcoreMesh(core_axis_name="c", subcore_axis_name="s"),
    compiler_params=pltpu.CompilerParams(needs_layout_passes=False),
    scratch_types=[
        pltpu.VMEM((TOT,), jnp.float32),
        pltpu.VMEM((TOT,), jnp.float32),
        pltpu.VMEM((TOT,), jnp.float32),
        pltpu.VMEM((TOT,), jnp.float32),
        pltpu.VMEM((NVREG,), jnp.float32),
        pltpu.VMEM((NVREG,), jnp.float32),
        pltpu.VMEM((NVREG,), jnp.int32),
        pltpu.VMEM((NVREG,), jnp.int32),
        pltpu.VMEM((TOPK,), jnp.int32),
        pltpu.VMEM((TOPK,), jnp.int32),
        pltpu.VMEM((2 * TOPK,), jnp.int32),
        pltpu.VMEM((128, ZD), jnp.float32),
        pltpu.SemaphoreType.DMA,
    ],
)
def _sel_kernel(att_hbm, z_hbm, mu_hbm, sig_hbm,
                ztopk_hbm, mutopk_hbm, sigtopk_hbm,
                att0, att1, key0, key1, l1k0, l1k1, l1p0, l1p1,
                pos0, pos1, idx_v, rows_v, sem):
    _sel_body(att_hbm, z_hbm, mu_hbm, sig_hbm,
              ztopk_hbm, mutopk_hbm, sigtopk_hbm,
              (att0, att1), (key0, key1), (l1k0, l1k1), (l1p0, l1p1),
              (pos0, pos1), idx_v, rows_v, sem)


def _dec_body(zs_ref, wd_ref, bd_ref, o_ref):
    acc = jnp.dot(zs_ref[...], wd_ref[...], preferred_element_type=jnp.float32)
    acc = acc + bd_ref[...]
    o_ref[...] = acc


def _decoder_logits(z_proxy_sample, Wd, bd):
    nproxy, S, zdim = z_proxy_sample.shape
    C = Wd.shape[1]
    flat = z_proxy_sample.reshape(nproxy * S, zdim)
    out = pl.pallas_call(
        _dec_body,
        out_shape=jax.ShapeDtypeStruct((nproxy * S, C), jnp.float32),
    )(flat, Wd, bd.reshape(1, -1))
    return jnp.mean(out.reshape(nproxy, S, C), axis=1)


def kernel(x, W1, b1, W2, b2, W3, b3, Wd, bd, proxies, eps_proxy, topk):
    zdim = W3.shape[1]

    mu_proxy = proxies[:, :zdim]
    sigma_proxy = jax.nn.softplus(proxies[:, zdim:])
    z_proxy_sample = mu_proxy[:, None, :] + sigma_proxy[:, None, :] * eps_proxy
    z_proxy = jnp.mean(z_proxy_sample, axis=1)

    def _l2norm(v, axis):
        n = jnp.sqrt(jnp.sum(v * v, axis=axis, keepdims=True))
        return v / jnp.maximum(n, 1e-12)

    z_proxy_norm = _l2norm(z_proxy, axis=1)

    h = jnp.maximum(jnp.einsum('bnf,fg->bng', x, W1) + b1, 0.0)
    h = jnp.maximum(jnp.einsum('bng,gh->bnh', h, W2) + b2, 0.0)
    z = jnp.einsum('bnh,hz->bnz', h, W3) + b3
    z_norm = _l2norm(z, axis=2)
    att = jnp.einsum('bnz,pz->bnp', z_norm, z_proxy_norm)

    z, att = jax.lax.optimization_barrier((z, att))

    att_t = jnp.transpose(att, (0, 2, 1)).reshape(B, TOT)
    z_flat = z.reshape(B * N, zdim)

    z_topk, mu_topk, sigma_topk = _sel_kernel(
        att_t, z_flat, mu_proxy, sigma_proxy)

    decoder_logits_proxy = _decoder_logits(z_proxy_sample, Wd, bd)
    return (decoder_logits_proxy, mu_proxy, sigma_proxy,
            z_topk, mu_topk, sigma_topk)


# speedup vs baseline: 1.0285x; 1.0243x over previous
import functools

import jax
import jax.numpy as jnp
from jax import lax
from jax.experimental import pallas as pl
from jax.experimental.pallas import tpu as pltpu
from jax.experimental.pallas import tpu_sc as plsc

B = 64
N = 512
P = 8
ZD = 256
TOPK = 256
TOT = N * P
NVREG = TOT // 16
import numpy as np

NEGINF = np.float32(-np.inf)
BIG = np.int32(1 << 30)


def _lanes():
    return lax.iota(jnp.int32, 16)


def _sel_body(att_hbm, z_hbm, mu_hbm, sig_hbm,
              ztopk_hbm, mutopk_hbm, sigtopk_hbm,
              att_v, key_v, key0_v, l1k_v, l1p_v, pos_v, selt_v, idx_v,
              rows_v, sem):
    wid = lax.axis_index("s") * 2 + lax.axis_index("c")
    lanes = _lanes()
    idx1_offs = (lanes % 8) * 512 + lanes // 8

    for bag_i in range(2):
        b = wid * 2 + bag_i

        pltpu.sync_copy(att_hbm.at[b], att_v)

        def p1(g, _):
            l1k = jnp.full((16,), NEGINF, jnp.float32)
            l1p = jnp.full((16,), BIG, jnp.int32)
            for j in range(16):
                i = g * 16 + j
                key = att_v[pl.ds(i * 16, 16)]
                key_v[pl.ds(i * 16, 16)] = key
                key0_v[pl.ds(i * 16, 16)] = key
                m = jnp.max(key)
                lp = jnp.min(jnp.where(key == m, lanes, jnp.int32(16)))
                l1k = jnp.where(lanes == j, m, l1k)
                l1p = jnp.where(lanes == j, i * 16 + lp, l1p)
            l1k_v[pl.ds(g * 16, 16)] = l1k
            l1p_v[pl.ds(g * 16, 16)] = l1p
            return 0

        lax.fori_loop(0, 16, p1, 0)

        def p2(g, carry):
            l2k, l2p = carry
            gk = l1k_v[pl.ds(g * 16, 16)]
            gp = l1p_v[pl.ds(g * 16, 16)]
            m = jnp.max(gk)
            pos = jnp.min(jnp.where(gk == m, gp, BIG))
            l2k = jnp.where(lanes == g, m, l2k)
            l2p = jnp.where(lanes == g, pos, l2p)
            return l2k, l2p

        l2k, l2p = lax.fori_loop(
            0, 16, p2,
            (jnp.full((16,), NEGINF, jnp.float32), jnp.full((16,), BIG, jnp.int32)))

        def p3(o, carry):
            l2k, l2p, tkey = carry
            posacc = jnp.full((16,), 0, jnp.int32)
            for j in range(16):
                m2 = jnp.max(l2k)
                win = jnp.min(jnp.where(l2k == m2, l2p, BIG))
                posacc = jnp.where(lanes == j, win, posacc)
                tkey = m2
                vidx = win // 16
                lane = win % 16
                vec = key_v[pl.ds(vidx * 16, 16)]
                vec = jnp.where(lanes == lane, NEGINF, vec)
                key_v[pl.ds(vidx * 16, 16)] = vec
                m1 = jnp.max(vec)
                p1n = vidx * 16 + jnp.min(jnp.where(vec == m1, lanes, jnp.int32(16)))
                grp = vidx // 16
                lane2 = vidx % 16
                gk = l1k_v[pl.ds(grp * 16, 16)]
                gp = l1p_v[pl.ds(grp * 16, 16)]
                gk = jnp.where(lanes == lane2, m1, gk)
                gp = jnp.where(lanes == lane2, jnp.where(m1 == NEGINF, BIG, p1n), gp)
                l1k_v[pl.ds(grp * 16, 16)] = gk
                l1p_v[pl.ds(grp * 16, 16)] = gp
                m2n = jnp.max(gk)
                p2n = jnp.min(jnp.where(gk == m2n, gp, BIG))
                l2k = jnp.where(lanes == grp, m2n, l2k)
                l2p = jnp.where(lanes == grp, p2n, l2p)
            pos_v[pl.ds(o * 16, 16)] = posacc
            return l2k, l2p, tkey

        _, _, tkey = lax.fori_loop(0, 16, p3, (l2k, l2p, NEGINF))

        def p4a(k, run):
            pos16 = idx1_offs + 2 * k
            key = plsc.load_gather(key0_v, [pos16])
            eq = key == tkey
            cum = lax.cumsum(jnp.where(eq, 1, 0).astype(jnp.int32), axis=0)
            sel = jnp.where(eq, (run + cum) <= r_allowed, False)
            plsc.store_scatter(selt_v, [pos16], jnp.where(sel, 1, 0).astype(jnp.int32))
            return run + jnp.max(cum)

        def cgt_body(i, acc):
            key = key0_v[pl.ds(i * 16, 16)]
            return acc + jnp.where(key > tkey, 1, 0).astype(jnp.int32)

        cgt_lanes = lax.fori_loop(0, NVREG, cgt_body,
                                  jnp.zeros((16,), jnp.int32))
        r_allowed = jnp.int32(TOPK) - jnp.sum(cgt_lanes)
        lax.fori_loop(0, NVREG, p4a, jnp.int32(0))

        cnts = jnp.where(lanes < P, 0, -1).astype(jnp.int32)
        for p in range(P):
            def prow(i, acc):
                key = key0_v[pl.ds(p * N + i * 16, 16)]
                s = selt_v[pl.ds(p * N + i * 16, 16)]
                return acc + jnp.where(key > tkey, 1, 0).astype(jnp.int32) + s

            acc = lax.fori_loop(0, N // 16, prow, jnp.zeros((16,), jnp.int32))
            cnts = jnp.where(lanes == p, jnp.sum(acc), cnts)
        mx = jnp.max(cnts)
        pstar = jnp.min(jnp.where(cnts == mx, lanes, jnp.int32(16)))

        pltpu.sync_copy(mu_hbm.at[pstar], mutopk_hbm.at[b])
        pltpu.sync_copy(sig_hbm.at[pstar], sigtopk_hbm.at[b])

        def p6(o, _):
            vec = pos_v[pl.ds(o * 16, 16)]
            idx_v[pl.ds(o * 16, 16)] = b * N + (vec & (N - 1))
            return 0

        lax.fori_loop(0, 16, p6, 0)
        for c in range(2):
            pltpu.async_copy(z_hbm.at[idx_v.at[pl.ds(c * 128, 128)]],
                             rows_v, sem).wait()
            pltpu.sync_copy(rows_v, ztopk_hbm.at[b, pl.ds(c * 128, 128)])


@functools.partial(
    pl.kernel,
    out_type=[
        jax.ShapeDtypeStruct((B, TOPK, ZD), jnp.float32),
        jax.ShapeDtypeStruct((B, ZD), jnp.float32),
        jax.ShapeDtypeStruct((B, ZD), jnp.float32),
    ],
    mesh=plsc.VectorSubcoreMesh(core_axis_name="c", subcore_axis_name="s"),
    compiler_params=pltpu.CompilerParams(needs_layout_passes=False),
    scratch_types=[
        pltpu.VMEM((TOT,), jnp.float32),
        pltpu.VMEM((TOT,), jnp.float32),
        pltpu.VMEM((TOT,), jnp.float32),
        pltpu.VMEM((NVREG,), jnp.float32),
        pltpu.VMEM((NVREG,), jnp.int32),
        pltpu.VMEM((TOPK,), jnp.int32),
        pltpu.VMEM((TOT,), jnp.int32),
        pltpu.VMEM((TOPK,), jnp.int32),
        pltpu.VMEM((128, ZD), jnp.float32),
        pltpu.SemaphoreType.DMA,
    ],
)
def _sel_kernel(att_hbm, z_hbm, mu_hbm, sig_hbm,
                ztopk_hbm, mutopk_hbm, sigtopk_hbm,
                att_v, key_v, key0_v, l1k_v, l1p_v, pos_v, selt_v, idx_v,
                rows_v, sem):
    _sel_body(att_hbm, z_hbm, mu_hbm, sig_hbm,
              ztopk_hbm, mutopk_hbm, sigtopk_hbm,
              att_v, key_v, key0_v, l1k_v, l1p_v, pos_v, selt_v, idx_v,
              rows_v, sem)


def _dec_body(zs_ref, wd_ref, bd_ref, o_ref):
    acc = jnp.dot(zs_ref[...], wd_ref[...], preferred_element_type=jnp.float32)
    acc = acc + bd_ref[...]
    o_ref[...] = acc


def _decoder_logits(z_proxy_sample, Wd, bd):
    nproxy, S, zdim = z_proxy_sample.shape
    C = Wd.shape[1]
    flat = z_proxy_sample.reshape(nproxy * S, zdim)
    out = pl.pallas_call(
        _dec_body,
        out_shape=jax.ShapeDtypeStruct((nproxy * S, C), jnp.float32),
    )(flat, Wd, bd.reshape(1, -1))
    return jnp.mean(out.reshape(nproxy, S, C), axis=1)


def kernel(x, W1, b1, W2, b2, W3, b3, Wd, bd, proxies, eps_proxy, topk):
    zdim = W3.shape[1]

    mu_proxy = proxies[:, :zdim]
    sigma_proxy = jax.nn.softplus(proxies[:, zdim:])
    z_proxy_sample = mu_proxy[:, None, :] + sigma_proxy[:, None, :] * eps_proxy
    z_proxy = jnp.mean(z_proxy_sample, axis=1)

    def _l2norm(v, axis):
        n = jnp.sqrt(jnp.sum(v * v, axis=axis, keepdims=True))
        return v / jnp.maximum(n, 1e-12)

    z_proxy_norm = _l2norm(z_proxy, axis=1)

    h = jnp.maximum(jnp.einsum('bnf,fg->bng', x, W1) + b1, 0.0)
    h = jnp.maximum(jnp.einsum('bng,gh->bnh', h, W2) + b2, 0.0)
    z = jnp.einsum('bnh,hz->bnz', h, W3) + b3
    z_norm = _l2norm(z, axis=2)
    att = jnp.einsum('bnz,pz->bnp', z_norm, z_proxy_norm)

    z, att = jax.lax.optimization_barrier((z, att))

    att_t = jnp.transpose(att, (0, 2, 1)).reshape(B, TOT)
    z_flat = z.reshape(B * N, zdim)

    z_topk, mu_topk, sigma_topk = _sel_kernel(
        att_t, z_flat, mu_proxy, sigma_proxy)

    decoder_logits_proxy = _decoder_logits(z_proxy_sample, Wd, bd)
    return (decoder_logits_proxy, mu_proxy, sigma_proxy,
            z_topk, mu_topk, sigma_topk)


# speedup vs baseline: 1.0478x; 1.0188x over previous
import functools

import jax
import jax.numpy as jnp
from jax import lax
from jax.experimental import pallas as pl
from jax.experimental.pallas import tpu as pltpu
from jax.experimental.pallas import tpu_sc as plsc

B = 64
N = 512
P = 8
ZD = 256
TOPK = 256
TOT = N * P
NVREG = TOT // 16
import numpy as np

NEGINF = np.float32(-np.inf)
BIG = np.int32(1 << 30)


def _lanes():
    return lax.iota(jnp.int32, 16)


def _sel_body(att_hbm, z_hbm, mu_hbm, sig_hbm,
              ztopk_hbm, mutopk_hbm, sigtopk_hbm,
              att_v, key_v, key0_v, l1k_v, l1p_v, pos_v, idx_v,
              rows_v, sem):
    wid = lax.axis_index("s") * 2 + lax.axis_index("c")
    lanes = _lanes()
    idx1_offs = (lanes % 8) * 512 + lanes // 8

    for bag_i in range(2):
        b = wid * 2 + bag_i

        pltpu.sync_copy(att_hbm.at[b], att_v)

        def p1(g, _):
            l1k = jnp.full((16,), NEGINF, jnp.float32)
            l1p = jnp.full((16,), BIG, jnp.int32)
            for j in range(16):
                i = g * 16 + j
                key = att_v[pl.ds(i * 16, 16)]
                key_v[pl.ds(i * 16, 16)] = key
                key0_v[pl.ds(i * 16, 16)] = key
                m = jnp.max(key)
                lp = jnp.min(jnp.where(key == m, lanes, jnp.int32(16)))
                l1k = jnp.where(lanes == j, m, l1k)
                l1p = jnp.where(lanes == j, i * 16 + lp, l1p)
            l1k_v[pl.ds(g * 16, 16)] = l1k
            l1p_v[pl.ds(g * 16, 16)] = l1p
            return 0

        lax.fori_loop(0, 16, p1, 0)

        def p2(g, carry):
            l2k, l2p = carry
            gk = l1k_v[pl.ds(g * 16, 16)]
            gp = l1p_v[pl.ds(g * 16, 16)]
            m = jnp.max(gk)
            pos = jnp.min(jnp.where(gk == m, gp, BIG))
            l2k = jnp.where(lanes == g, m, l2k)
            l2p = jnp.where(lanes == g, pos, l2p)
            return l2k, l2p

        l2k, l2p = lax.fori_loop(
            0, 16, p2,
            (jnp.full((16,), NEGINF, jnp.float32), jnp.full((16,), BIG, jnp.int32)))

        def p3(o, carry):
            l2k, l2p, tkey = carry
            posacc = jnp.full((16,), 0, jnp.int32)
            for j in range(16):
                m2 = jnp.max(l2k)
                win = jnp.min(jnp.where(l2k == m2, l2p, BIG))
                posacc = jnp.where(lanes == j, win, posacc)
                tkey = m2
                vidx = win // 16
                lane = win % 16
                vec = key_v[pl.ds(vidx * 16, 16)]
                vec = jnp.where(lanes == lane, NEGINF, vec)
                key_v[pl.ds(vidx * 16, 16)] = vec
                m1 = jnp.max(vec)
                p1n = vidx * 16 + jnp.min(jnp.where(vec == m1, lanes, jnp.int32(16)))
                grp = vidx // 16
                lane2 = vidx % 16
                gk = l1k_v[pl.ds(grp * 16, 16)]
                gp = l1p_v[pl.ds(grp * 16, 16)]
                gk = jnp.where(lanes == lane2, m1, gk)
                gp = jnp.where(lanes == lane2, jnp.where(m1 == NEGINF, BIG, p1n), gp)
                l1k_v[pl.ds(grp * 16, 16)] = gk
                l1p_v[pl.ds(grp * 16, 16)] = gp
                m2n = jnp.max(gk)
                p2n = jnp.min(jnp.where(gk == m2n, gp, BIG))
                l2k = jnp.where(lanes == grp, m2n, l2k)
                l2p = jnp.where(lanes == grp, p2n, l2p)
            pos_v[pl.ds(o * 16, 16)] = posacc
            return l2k, l2p, tkey

        _, _, tkey = lax.fori_loop(0, 16, p3, (l2k, l2p, NEGINF))

        def p4a(k, carry):
            run, tie_lanes = carry
            pos16 = idx1_offs + 2 * k
            key = plsc.load_gather(key0_v, [pos16])
            eq = key == tkey
            cum = lax.cumsum(jnp.where(eq, 1, 0).astype(jnp.int32), axis=0)
            sel = jnp.where(eq, (run + cum) <= r_allowed, False)
            tie_lanes = tie_lanes + jnp.where(sel, 1, 0).astype(jnp.int32)
            return run + jnp.max(cum), tie_lanes

        def cgt_body(i, acc):
            key = key0_v[pl.ds(i * 16, 16)]
            return acc + jnp.where(key > tkey, 1, 0).astype(jnp.int32)

        cgt_lanes = lax.fori_loop(0, NVREG, cgt_body,
                                  jnp.zeros((16,), jnp.int32))
        r_allowed = jnp.int32(TOPK) - jnp.sum(cgt_lanes)
        _, tie_lanes = lax.fori_loop(
            0, NVREG, p4a, (jnp.int32(0), jnp.zeros((16,), jnp.int32)))

        cnts = jnp.where(lanes < P, 0, -1).astype(jnp.int32)
        for p in range(P):
            def prow(i, acc):
                key = key0_v[pl.ds(p * N + i * 16, 16)]
                return acc + jnp.where(key > tkey, 1, 0).astype(jnp.int32)

            acc = lax.fori_loop(0, N // 16, prow, jnp.zeros((16,), jnp.int32))
            tp = jnp.sum(jnp.where((lanes == p) | (lanes == p + 8), tie_lanes, 0))
            cnts = jnp.where(lanes == p, jnp.sum(acc) + tp, cnts)
        mx = jnp.max(cnts)
        pstar = jnp.min(jnp.where(cnts == mx, lanes, jnp.int32(16)))

        pltpu.sync_copy(mu_hbm.at[pstar], mutopk_hbm.at[b])
        pltpu.sync_copy(sig_hbm.at[pstar], sigtopk_hbm.at[b])

        def p6(o, _):
            vec = pos_v[pl.ds(o * 16, 16)]
            idx_v[pl.ds(o * 16, 16)] = b * N + (vec & (N - 1))
            return 0

        lax.fori_loop(0, 16, p6, 0)
        for c in range(2):
            pltpu.async_copy(z_hbm.at[idx_v.at[pl.ds(c * 128, 128)]],
                             rows_v, sem).wait()
            pltpu.sync_copy(rows_v, ztopk_hbm.at[b, pl.ds(c * 128, 128)])


@functools.partial(
    pl.kernel,
    out_type=[
        jax.ShapeDtypeStruct((B, TOPK, ZD), jnp.float32),
        jax.ShapeDtypeStruct((B, ZD), jnp.float32),
        jax.ShapeDtypeStruct((B, ZD), jnp.float32),
    ],
    mesh=plsc.VectorSubcoreMesh(core_axis_name="c", subcore_axis_name="s"),
    compiler_params=pltpu.CompilerParams(needs_layout_passes=False),
    scratch_types=[
        pltpu.VMEM((TOT,), jnp.float32),
        pltpu.VMEM((TOT,), jnp.float32),
        pltpu.VMEM((TOT,), jnp.float32),
        pltpu.VMEM((NVREG,), jnp.float32),
        pltpu.VMEM((NVREG,), jnp.int32),
        pltpu.VMEM((TOPK,), jnp.int32),
        pltpu.VMEM((TOPK,), jnp.int32),
        pltpu.VMEM((128, ZD), jnp.float32),
        pltpu.SemaphoreType.DMA,
    ],
)
def _sel_kernel(att_hbm, z_hbm, mu_hbm, sig_hbm,
                ztopk_hbm, mutopk_hbm, sigtopk_hbm,
                att_v, key_v, key0_v, l1k_v, l1p_v, pos_v, idx_v,
                rows_v, sem):
    _sel_body(att_hbm, z_hbm, mu_hbm, sig_hbm,
              ztopk_hbm, mutopk_hbm, sigtopk_hbm,
              att_v, key_v, key0_v, l1k_v, l1p_v, pos_v, idx_v,
              rows_v, sem)


def _dec_body(zs_ref, wd_ref, bd_ref, o_ref):
    acc = jnp.dot(zs_ref[...], wd_ref[...], preferred_element_type=jnp.float32)
    acc = acc + bd_ref[...]
    o_ref[...] = acc


def _decoder_logits(z_proxy_sample, Wd, bd):
    nproxy, S, zdim = z_proxy_sample.shape
    C = Wd.shape[1]
    flat = z_proxy_sample.reshape(nproxy * S, zdim)
    out = pl.pallas_call(
        _dec_body,
        out_shape=jax.ShapeDtypeStruct((nproxy * S, C), jnp.float32),
    )(flat, Wd, bd.reshape(1, -1))
    return jnp.mean(out.reshape(nproxy, S, C), axis=1)


def kernel(x, W1, b1, W2, b2, W3, b3, Wd, bd, proxies, eps_proxy, topk):
    zdim = W3.shape[1]

    mu_proxy = proxies[:, :zdim]
    sigma_proxy = jax.nn.softplus(proxies[:, zdim:])
    z_proxy_sample = mu_proxy[:, None, :] + sigma_proxy[:, None, :] * eps_proxy
    z_proxy = jnp.mean(z_proxy_sample, axis=1)

    def _l2norm(v, axis):
        n = jnp.sqrt(jnp.sum(v * v, axis=axis, keepdims=True))
        return v / jnp.maximum(n, 1e-12)

    z_proxy_norm = _l2norm(z_proxy, axis=1)

    h = jnp.maximum(jnp.einsum('bnf,fg->bng', x, W1) + b1, 0.0)
    h = jnp.maximum(jnp.einsum('bng,gh->bnh', h, W2) + b2, 0.0)
    z = jnp.einsum('bnh,hz->bnz', h, W3) + b3
    z_norm = _l2norm(z, axis=2)
    att = jnp.einsum('bnz,pz->bnp', z_norm, z_proxy_norm)

    z, att = jax.lax.optimization_barrier((z, att))

    att_t = jnp.transpose(att, (0, 2, 1)).reshape(B, TOT)
    z_flat = z.reshape(B * N, zdim)

    z_topk, mu_topk, sigma_topk = _sel_kernel(
        att_t, z_flat, mu_proxy, sigma_proxy)

    decoder_logits_proxy = _decoder_logits(z_proxy_sample, Wd, bd)
    return (decoder_logits_proxy, mu_proxy, sigma_proxy,
            z_topk, mu_topk, sigma_topk)
